# Initial kernel scaffold; baseline (speedup 1.0000x reference)
import jax
import jax.numpy as jnp
from jax.experimental import pallas as pl


def kernel(x, edge_index, edge_attr, W, b):
    raise NotImplementedError("write your pallas kernel here")


# speedup vs baseline: 4.8240x; 4.8240x over previous
import functools

import jax
import jax.numpy as jnp
from jax import lax
from jax.experimental import pallas as pl
from jax.experimental.pallas import tpu as pltpu
from jax.experimental.pallas import tpu_sc as plsc

N_NODES = 10000
N_EDGES = 320000
D_FEAT = 128
D_EDGE = 16
D_OUT = 16

NC, NS = 2, 16
NW = NC * NS
E_PER_W = N_EDGES // NW
CHUNK = 2000
N_CHUNKS = E_PER_W // CHUNK

BASE_BLK = 4000


def _proj_body(x_ref, w_ref, xs_ref, xr_ref):
    p = jnp.dot(x_ref[...], w_ref[...], preferred_element_type=jnp.float32)
    xs_ref[...] = p[:, :D_OUT]
    xr_ref[...] = p[:, D_OUT:]


def _node_proj(x, w_sr):
    return pl.pallas_call(
        _proj_body,
        out_shape=(
            jax.ShapeDtypeStruct((N_NODES, D_OUT), jnp.float32),
            jax.ShapeDtypeStruct((N_NODES, D_OUT), jnp.float32),
        ),
    )(x, w_sr)


def _base_body(ea_ref, bd_ref, b_ref, o_ref):
    o_ref[...] = (
        jnp.dot(ea_ref[...], bd_ref[...], preferred_element_type=jnp.float32)
        + b_ref[...]
    )


def _edge_base(ea8, bd, b8):
    n_rows = N_EDGES // 8
    return pl.pallas_call(
        _base_body,
        grid=(n_rows // BASE_BLK,),
        in_specs=[
            pl.BlockSpec((BASE_BLK, 128), lambda i: (i, 0)),
            pl.BlockSpec((128, 128), lambda i: (0, 0)),
            pl.BlockSpec((1, 128), lambda i: (0, 0)),
        ],
        out_specs=pl.BlockSpec((BASE_BLK, 128), lambda i: (i, 0)),
        out_shape=jax.ShapeDtypeStruct((n_rows, 128), jnp.float32),
    )(ea8, bd, b8)


_SC_MESH = plsc.VectorSubcoreMesh(core_axis_name="c", subcore_axis_name="s")


@functools.partial(
    pl.kernel,
    out_type=jax.ShapeDtypeStruct((N_EDGES, D_OUT), jnp.float32),
    mesh=_SC_MESH,
    compiler_params=pltpu.CompilerParams(use_tc_tiling_on_sc=False),
    scratch_types=[
        pltpu.VMEM((CHUNK,), jnp.int32),
        pltpu.VMEM((CHUNK,), jnp.int32),
        pltpu.VMEM((CHUNK, D_OUT), jnp.float32),
        pltpu.SemaphoreType.DMA,
        pltpu.SemaphoreType.DMA,
    ],
)
def _sc_edge(xs_hbm, xr_hbm, sidx_hbm, ridx_hbm, base_hbm, out_hbm,
             idx_s, idx_r, buf, sem0, sem1):
    wid = lax.axis_index("s") * NC + lax.axis_index("c")
    row0 = wid * E_PER_W

    def chunk_body(ci, carry):
        off = row0 + ci * CHUNK
        pltpu.sync_copy(sidx_hbm.at[pl.ds(off, CHUNK)], idx_s)
        pltpu.sync_copy(ridx_hbm.at[pl.ds(off, CHUNK)], idx_r)
        pltpu.sync_copy(base_hbm.at[pl.ds(off, CHUNK)], buf)
        pltpu.async_copy(xs_hbm.at[idx_s], buf, sem0, add=True).wait()
        pltpu.async_copy(xr_hbm.at[idx_r], buf, sem1, add=True).wait()
        pltpu.sync_copy(buf, out_hbm.at[pl.ds(off, CHUNK)])
        return carry

    lax.fori_loop(0, N_CHUNKS, chunk_body, 0)


def kernel(x, edge_index, edge_attr, W, b):
    w_sr = jnp.concatenate([W[:D_FEAT], W[D_FEAT:2 * D_FEAT]], axis=1)
    bd = jnp.kron(jnp.eye(8, dtype=W.dtype), W[2 * D_FEAT:])
    b8 = jnp.tile(b, 8).reshape(1, 128)

    xs, xr = _node_proj(x, w_sr)
    base8 = _edge_base(edge_attr.reshape(N_EDGES // 8, 128), bd, b8)
    base = base8.reshape(N_EDGES, D_OUT)

    return _sc_edge(xs, xr, edge_index[0], edge_index[1], base)


# speedup vs baseline: 6.9458x; 1.4398x over previous
import functools

import jax
import jax.numpy as jnp
from jax import lax
from jax.experimental import pallas as pl
from jax.experimental.pallas import tpu as pltpu
from jax.experimental.pallas import tpu_sc as plsc

N_NODES = 10000
N_EDGES = 320000
D_FEAT = 128
D_EDGE = 16
D_OUT = 16

NC, NS = 2, 16
NW = NC * NS
E_PER_W = N_EDGES // NW
CHUNK = 2000
N_CHUNKS = E_PER_W // CHUNK

FIN_BLK = 16000


def _proj_body(x_ref, w_ref, xs_ref, xr_ref):
    p = jnp.dot(x_ref[...], w_ref[...], preferred_element_type=jnp.float32)
    xs_ref[...] = p[:, :D_OUT]
    xr_ref[...] = p[:, D_OUT:]


def _node_proj(x, w_sr):
    return pl.pallas_call(
        _proj_body,
        out_shape=(
            jax.ShapeDtypeStruct((N_NODES, D_OUT), jnp.float32),
            jax.ShapeDtypeStruct((N_NODES, D_OUT), jnp.float32),
        ),
    )(x, w_sr)


def _final_body(eat_ref, tmpt_ref, wet_ref, b_ref, o_ref):
    o_ref[...] = (
        jnp.dot(wet_ref[...], eat_ref[...], preferred_element_type=jnp.float32)
        + b_ref[...]
        + tmpt_ref[...]
    )


def _final(eat, tmpt, wet, bcol):
    return pl.pallas_call(
        _final_body,
        grid=(N_EDGES // FIN_BLK,),
        in_specs=[
            pl.BlockSpec((D_OUT, FIN_BLK), lambda i: (0, i)),
            pl.BlockSpec((D_OUT, FIN_BLK), lambda i: (0, i)),
            pl.BlockSpec((D_EDGE, D_OUT), lambda i: (0, 0)),
            pl.BlockSpec((D_OUT, 1), lambda i: (0, 0)),
        ],
        out_specs=pl.BlockSpec((D_OUT, FIN_BLK), lambda i: (0, i)),
        out_shape=jax.ShapeDtypeStruct((D_OUT, N_EDGES), jnp.float32),
    )(eat, tmpt, wet, bcol)


_SC_MESH = plsc.VectorSubcoreMesh(core_axis_name="c", subcore_axis_name="s")


@functools.partial(
    pl.kernel,
    out_type=jax.ShapeDtypeStruct((N_EDGES, D_OUT), jnp.float32),
    mesh=_SC_MESH,
    compiler_params=pltpu.CompilerParams(use_tc_tiling_on_sc=False),
    scratch_types=[
        pltpu.VMEM((CHUNK,), jnp.int32),
        pltpu.VMEM((CHUNK,), jnp.int32),
        pltpu.VMEM((CHUNK, D_OUT), jnp.float32),
        pltpu.SemaphoreType.DMA,
        pltpu.SemaphoreType.DMA,
    ],
)
def _sc_edge(xs_hbm, xr_hbm, sidx_hbm, ridx_hbm, out_hbm,
             idx_s, idx_r, buf, sem0, sem1):
    wid = lax.axis_index("s") * NC + lax.axis_index("c")
    row0 = wid * E_PER_W

    def chunk_body(ci, carry):
        off = row0 + ci * CHUNK
        pltpu.sync_copy(sidx_hbm.at[pl.ds(off, CHUNK)], idx_s)
        pltpu.sync_copy(ridx_hbm.at[pl.ds(off, CHUNK)], idx_r)
        pltpu.async_copy(xs_hbm.at[idx_s], buf, sem0).wait()
        pltpu.async_copy(xr_hbm.at[idx_r], buf, sem1, add=True).wait()
        pltpu.sync_copy(buf, out_hbm.at[pl.ds(off, CHUNK)])
        return carry

    lax.fori_loop(0, N_CHUNKS, chunk_body, 0)


def kernel(x, edge_index, edge_attr, W, b):
    w_sr = jnp.concatenate([W[:D_FEAT], W[D_FEAT:2 * D_FEAT]], axis=1)
    wet = W[2 * D_FEAT:].T
    bcol = b.reshape(D_OUT, 1)

    xs, xr = _node_proj(x, w_sr)
    tmp = _sc_edge(xs, xr, edge_index[0], edge_index[1])
    out_t = _final(edge_attr.T, tmp.T, wet, bcol)
    return out_t.T


# speedup vs baseline: 8.3732x; 1.2055x over previous
import functools

import jax
import jax.numpy as jnp
from jax import lax
from jax.experimental import pallas as pl
from jax.experimental.pallas import tpu as pltpu
from jax.experimental.pallas import tpu_sc as plsc

N_NODES = 10000
N_EDGES = 320000
D_FEAT = 128
D_EDGE = 16
D_OUT = 16

NC, NS = 2, 16
NW = NC * NS
E_PER_W = N_EDGES // NW
CHUNK = 2000
N_CHUNKS = E_PER_W // CHUNK

FIN_BLK = 16000


def _proj_body(x_ref, w_ref, xs_ref, xr_ref):
    p = jnp.dot(x_ref[...], w_ref[...], preferred_element_type=jnp.float32)
    xs_ref[...] = p[:, :D_OUT]
    xr_ref[...] = p[:, D_OUT:]


def _node_proj(x, w_sr):
    return pl.pallas_call(
        _proj_body,
        out_shape=(
            jax.ShapeDtypeStruct((N_NODES, D_OUT), jnp.float32),
            jax.ShapeDtypeStruct((N_NODES, D_OUT), jnp.float32),
        ),
    )(x, w_sr)


def _final_body(eat_ref, tmpt_ref, wet_ref, b_ref, o_ref):
    o_ref[...] = (
        jnp.dot(wet_ref[...], eat_ref[...], preferred_element_type=jnp.float32)
        + b_ref[...]
        + tmpt_ref[...]
    )


def _final(eat, tmpt, wet, bcol):
    return pl.pallas_call(
        _final_body,
        grid=(N_EDGES // FIN_BLK,),
        in_specs=[
            pl.BlockSpec((D_OUT, FIN_BLK), lambda i: (0, i)),
            pl.BlockSpec((D_OUT, FIN_BLK), lambda i: (0, i)),
            pl.BlockSpec((D_EDGE, D_OUT), lambda i: (0, 0)),
            pl.BlockSpec((D_OUT, 1), lambda i: (0, 0)),
        ],
        out_specs=pl.BlockSpec((D_OUT, FIN_BLK), lambda i: (0, i)),
        out_shape=jax.ShapeDtypeStruct((D_OUT, N_EDGES), jnp.float32),
    )(eat, tmpt, wet, bcol)


_SC_MESH = plsc.VectorSubcoreMesh(core_axis_name="c", subcore_axis_name="s")


@functools.partial(
    pl.kernel,
    out_type=jax.ShapeDtypeStruct((D_OUT, N_EDGES), jnp.float32),
    mesh=_SC_MESH,
    compiler_params=pltpu.CompilerParams(
        use_tc_tiling_on_sc=False, needs_layout_passes=False),
    scratch_types=[
        pltpu.VMEM((CHUNK,), jnp.int32),
        pltpu.VMEM((CHUNK,), jnp.int32),
        pltpu.VMEM((CHUNK, D_OUT), jnp.float32),
        pltpu.VMEM((D_OUT, CHUNK), jnp.float32),
        pltpu.SemaphoreType.DMA,
        pltpu.SemaphoreType.DMA,
    ],
)
def _sc_edge(xs_hbm, xr_hbm, sidx_hbm, ridx_hbm, out_hbm,
             idx_s, idx_r, buf, buft, sem0, sem1):
    wid = lax.axis_index("s") * NC + lax.axis_index("c")
    row0 = wid * E_PER_W
    lane = lax.iota(jnp.int32, 16)

    def chunk_body(ci, carry):
        off = row0 + ci * CHUNK
        pltpu.sync_copy(sidx_hbm.at[pl.ds(off, CHUNK)], idx_s)
        pltpu.sync_copy(ridx_hbm.at[pl.ds(off, CHUNK)], idx_r)
        pltpu.async_copy(xs_hbm.at[idx_s], buf, sem0).wait()
        pltpu.async_copy(xr_hbm.at[idx_r], buf, sem1, add=True).wait()

        def t_body(g, c):
            rows = g * 16 + lane
            for j in range(D_OUT):
                v = plsc.load_gather(buf, [rows, jnp.full((16,), j, jnp.int32)])
                buft[j, pl.ds(g * 16, 16)] = v
            return c

        lax.fori_loop(0, CHUNK // 16, t_body, 0)
        for j in range(D_OUT):
            pltpu.sync_copy(buft.at[j], out_hbm.at[j, pl.ds(off, CHUNK)])
        return carry

    lax.fori_loop(0, N_CHUNKS, chunk_body, 0)


def kernel(x, edge_index, edge_attr, W, b):
    w_sr = jnp.concatenate([W[:D_FEAT], W[D_FEAT:2 * D_FEAT]], axis=1)
    wet = W[2 * D_FEAT:].T
    bcol = b.reshape(D_OUT, 1)

    xs, xr = _node_proj(x, w_sr)
    tmpt = _sc_edge(xs, xr, edge_index[0], edge_index[1])
    out_t = _final(edge_attr.T, tmpt, wet, bcol)
    return out_t.T


# speedup vs baseline: 11.7235x; 1.4001x over previous
import functools

import jax
import jax.numpy as jnp
from jax import lax
from jax.experimental import pallas as pl
from jax.experimental.pallas import tpu as pltpu
from jax.experimental.pallas import tpu_sc as plsc

N_NODES = 10000
N_EDGES = 320000
D_FEAT = 128
D_EDGE = 16
D_OUT = 16

NC, NS = 2, 16
NW = NC * NS
E_PER_W = N_EDGES // NW
CHUNK = 2000
N_CHUNKS = E_PER_W // CHUNK

FIN_BLK = 16000


def _proj_body(x_ref, w_ref, xs_ref, xr_ref):
    p = jnp.dot(x_ref[...], w_ref[...], preferred_element_type=jnp.float32)
    xs_ref[...] = p[:, :D_OUT]
    xr_ref[...] = p[:, D_OUT:]


def _node_proj(x, w_sr):
    return pl.pallas_call(
        _proj_body,
        out_shape=(
            jax.ShapeDtypeStruct((N_NODES, D_OUT), jnp.float32),
            jax.ShapeDtypeStruct((N_NODES, D_OUT), jnp.float32),
        ),
    )(x, w_sr)


def _final_body(eat_ref, tmpt_ref, wet_ref, b_ref, o_ref):
    o_ref[...] = (
        jnp.dot(wet_ref[...], eat_ref[...], preferred_element_type=jnp.float32)
        + b_ref[...]
        + tmpt_ref[...]
    )


def _final(eat, tmpt, wet, bcol):
    return pl.pallas_call(
        _final_body,
        grid=(N_EDGES // FIN_BLK,),
        in_specs=[
            pl.BlockSpec((D_OUT, FIN_BLK), lambda i: (0, i)),
            pl.BlockSpec((D_OUT, FIN_BLK), lambda i: (0, i)),
            pl.BlockSpec((D_EDGE, D_OUT), lambda i: (0, 0)),
            pl.BlockSpec((D_OUT, 1), lambda i: (0, 0)),
        ],
        out_specs=pl.BlockSpec((D_OUT, FIN_BLK), lambda i: (0, i)),
        out_shape=jax.ShapeDtypeStruct((D_OUT, N_EDGES), jnp.float32),
    )(eat, tmpt, wet, bcol)


_SC_MESH = plsc.VectorSubcoreMesh(core_axis_name="c", subcore_axis_name="s")


@functools.partial(
    pl.kernel,
    out_type=jax.ShapeDtypeStruct((D_OUT, N_EDGES), jnp.float32),
    mesh=_SC_MESH,
    compiler_params=pltpu.CompilerParams(
        use_tc_tiling_on_sc=False, needs_layout_passes=False),
    scratch_types=[
        pltpu.VMEM((CHUNK,), jnp.int32),
        pltpu.VMEM((CHUNK,), jnp.int32),
        pltpu.VMEM((CHUNK, D_OUT), jnp.float32),
        pltpu.VMEM((D_OUT, CHUNK), jnp.float32),
        pltpu.SemaphoreType.DMA,
        pltpu.SemaphoreType.DMA,
    ],
)
def _sc_edge(xs_hbm, xr_hbm, sidx_hbm, ridx_hbm, out_hbm,
             idx_s, idx_r, buf, buft, sem0, sem1):
    wid = lax.axis_index("s") * NC + lax.axis_index("c")
    row0 = wid * E_PER_W
    lane = lax.iota(jnp.int32, 16)
    diags = [(lane + j) % 16 for j in range(D_OUT)]

    def chunk_body(ci, carry):
        off = row0 + ci * CHUNK
        pltpu.sync_copy(sidx_hbm.at[pl.ds(off, CHUNK)], idx_s)
        pltpu.sync_copy(ridx_hbm.at[pl.ds(off, CHUNK)], idx_r)
        pltpu.async_copy(xs_hbm.at[idx_s], buf, sem0).wait()
        pltpu.async_copy(xr_hbm.at[idx_r], buf, sem1, add=True).wait()

        def t_body(g, c):
            rows = g * 16 + lane
            vs = [plsc.load_gather(buf, [rows, d]) for d in diags]
            for j in range(D_OUT):
                plsc.store_scatter(buft, [diags[j], rows], vs[j])
            return c

        lax.fori_loop(0, CHUNK // 16, t_body, 0)
        for j in range(D_OUT):
            pltpu.sync_copy(buft.at[j], out_hbm.at[j, pl.ds(off, CHUNK)])
        return carry

    lax.fori_loop(0, N_CHUNKS, chunk_body, 0)


def kernel(x, edge_index, edge_attr, W, b):
    w_sr = jnp.concatenate([W[:D_FEAT], W[D_FEAT:2 * D_FEAT]], axis=1)
    wet = W[2 * D_FEAT:].T
    bcol = b.reshape(D_OUT, 1)

    xs, xr = _node_proj(x, w_sr)
    tmpt = _sc_edge(xs, xr, edge_index[0], edge_index[1])
    out_t = _final(edge_attr.T, tmpt, wet, bcol)
    return out_t.T


# speedup vs baseline: 12.4988x; 1.0661x over previous
import functools

import jax
import jax.numpy as jnp
from jax import lax
from jax.experimental import pallas as pl
from jax.experimental.pallas import tpu as pltpu
from jax.experimental.pallas import tpu_sc as plsc

N_NODES = 10000
N_EDGES = 320000
D_FEAT = 128
D_EDGE = 16
D_OUT = 16

NC, NS = 2, 16
NW = NC * NS
E_PER_W = N_EDGES // NW
CHUNK = 2000
N_CHUNKS = E_PER_W // CHUNK

FIN_BLK = 16000


def _proj_body(x_ref, w_ref, xs_ref, xr_ref):
    p = jnp.dot(x_ref[...], w_ref[...], preferred_element_type=jnp.float32)
    xs_ref[...] = p[:, :D_OUT]
    xr_ref[...] = p[:, D_OUT:]


def _node_proj(x, w_sr):
    return pl.pallas_call(
        _proj_body,
        out_shape=(
            jax.ShapeDtypeStruct((N_NODES, D_OUT), jnp.float32),
            jax.ShapeDtypeStruct((N_NODES, D_OUT), jnp.float32),
        ),
    )(x, w_sr)


def _final_body(eat_ref, tmpt_ref, wet_ref, b_ref, o_ref):
    o_ref[...] = (
        jnp.dot(wet_ref[...], eat_ref[...], preferred_element_type=jnp.float32)
        + b_ref[...]
        + tmpt_ref[...]
    )


def _final(eat, tmpt, wet, bcol):
    return pl.pallas_call(
        _final_body,
        grid=(N_EDGES // FIN_BLK,),
        in_specs=[
            pl.BlockSpec((D_OUT, FIN_BLK), lambda i: (0, i)),
            pl.BlockSpec((D_OUT, FIN_BLK), lambda i: (0, i)),
            pl.BlockSpec((D_EDGE, D_OUT), lambda i: (0, 0)),
            pl.BlockSpec((D_OUT, 1), lambda i: (0, 0)),
        ],
        out_specs=pl.BlockSpec((D_OUT, FIN_BLK), lambda i: (0, i)),
        out_shape=jax.ShapeDtypeStruct((D_OUT, N_EDGES), jnp.float32),
    )(eat, tmpt, wet, bcol)


_SC_MESH = plsc.VectorSubcoreMesh(core_axis_name="c", subcore_axis_name="s")


@functools.partial(
    pl.kernel,
    out_type=jax.ShapeDtypeStruct((D_OUT, N_EDGES), jnp.float32),
    mesh=_SC_MESH,
    compiler_params=pltpu.CompilerParams(
        use_tc_tiling_on_sc=False, needs_layout_passes=False),
    scratch_types=[
        pltpu.VMEM((CHUNK,), jnp.int32),
        pltpu.VMEM((CHUNK,), jnp.int32),
        pltpu.VMEM((CHUNK, D_OUT), jnp.float32),
        pltpu.VMEM((D_OUT, CHUNK), jnp.float32),
        pltpu.SemaphoreType.DMA,
        pltpu.SemaphoreType.DMA,
    ],
)
def _sc_edge(xs_hbm, xr_hbm, eidx_hbm, out_hbm,
             idx_s, idx_r, buf, buft, sem0, sem1):
    wid = lax.axis_index("s") * NC + lax.axis_index("c")
    row0 = wid * E_PER_W
    lane = lax.iota(jnp.int32, 16)
    diags = [(lane + j) % 16 for j in range(D_OUT)]

    def chunk_body(ci, carry):
        off = row0 + ci * CHUNK
        pltpu.sync_copy(eidx_hbm.at[0, pl.ds(off, CHUNK)], idx_s)
        pltpu.sync_copy(eidx_hbm.at[1, pl.ds(off, CHUNK)], idx_r)
        pltpu.async_copy(xs_hbm.at[idx_s], buf, sem0).wait()
        pltpu.async_copy(xr_hbm.at[idx_r], buf, sem1, add=True).wait()

        def t_body(g, c):
            rows = g * 16 + lane
            vs = [plsc.load_gather(buf, [rows, d]) for d in diags]
            for j in range(D_OUT):
                plsc.store_scatter(buft, [diags[j], rows], vs[j])
            return c

        lax.fori_loop(0, CHUNK // 16, t_body, 0)
        for j in range(D_OUT):
            pltpu.sync_copy(buft.at[j], out_hbm.at[j, pl.ds(off, CHUNK)])
        return carry

    lax.fori_loop(0, N_CHUNKS, chunk_body, 0)


def kernel(x, edge_index, edge_attr, W, b):
    w_sr = jnp.concatenate([W[:D_FEAT], W[D_FEAT:2 * D_FEAT]], axis=1)
    wet = W[2 * D_FEAT:].T
    bcol = b.reshape(D_OUT, 1)

    xs, xr = _node_proj(x, w_sr)
    tmpt = _sc_edge(xs, xr, edge_index)
    out_t = _final(edge_attr.T, tmpt, wet, bcol)
    return out_t.T


# speedup vs baseline: 13.1919x; 1.0555x over previous
import functools

import jax
import jax.numpy as jnp
from jax import lax
from jax.experimental import pallas as pl
from jax.experimental.pallas import tpu as pltpu
from jax.experimental.pallas import tpu_sc as plsc

N_NODES = 10000
N_EDGES = 320000
D_FEAT = 128
D_EDGE = 16
D_OUT = 16

NC, NS = 2, 16
NW = NC * NS
E_PER_W = N_EDGES // NW
CHUNK = 2000
N_CHUNKS = E_PER_W // CHUNK

FIN_BLK = 16000


def _proj_body(x_ref, w_ref, xs_ref, xr_ref):
    p = jnp.dot(x_ref[...], w_ref[...], preferred_element_type=jnp.float32)
    xs_ref[...] = p[:, :D_OUT]
    xr_ref[...] = p[:, D_OUT:]


def _node_proj(x, w_sr):
    return pl.pallas_call(
        _proj_body,
        out_shape=(
            jax.ShapeDtypeStruct((N_NODES, D_OUT), jnp.float32),
            jax.ShapeDtypeStruct((N_NODES, D_OUT), jnp.float32),
        ),
    )(x, w_sr)


def _final_body(eat_ref, tmpt_ref, wet_ref, b_ref, o_ref):
    o_ref[...] = (
        jnp.dot(wet_ref[...], eat_ref[...], preferred_element_type=jnp.float32)
        + b_ref[...]
        + tmpt_ref[...]
    )


def _final(eat, tmpt, wet, bcol):
    return pl.pallas_call(
        _final_body,
        grid=(N_EDGES // FIN_BLK,),
        in_specs=[
            pl.BlockSpec((D_OUT, FIN_BLK), lambda i: (0, i)),
            pl.BlockSpec((D_OUT, FIN_BLK), lambda i: (0, i)),
            pl.BlockSpec((D_EDGE, D_OUT), lambda i: (0, 0)),
            pl.BlockSpec((D_OUT, 1), lambda i: (0, 0)),
        ],
        out_specs=pl.BlockSpec((D_OUT, FIN_BLK), lambda i: (0, i)),
        out_shape=jax.ShapeDtypeStruct((D_OUT, N_EDGES), jnp.float32),
    )(eat, tmpt, wet, bcol)


_SC_MESH = plsc.VectorSubcoreMesh(core_axis_name="c", subcore_axis_name="s")


@functools.partial(
    pl.kernel,
    out_type=jax.ShapeDtypeStruct((D_OUT, N_EDGES), jnp.float32),
    mesh=_SC_MESH,
    compiler_params=pltpu.CompilerParams(
        use_tc_tiling_on_sc=False, needs_layout_passes=False),
    scratch_types=[
        pltpu.VMEM((2, CHUNK), jnp.int32),
        pltpu.VMEM((2, CHUNK), jnp.int32),
        pltpu.VMEM((CHUNK, D_OUT), jnp.float32),
        pltpu.VMEM((CHUNK, D_OUT), jnp.float32),
        pltpu.VMEM((D_OUT, CHUNK), jnp.float32),
        pltpu.SemaphoreType.DMA,
        pltpu.SemaphoreType.DMA,
        pltpu.SemaphoreType.DMA,
    ],
)
def _sc_edge(xs_hbm, xr_hbm, eidx_hbm, out_hbm,
             idx_s, idx_r, bufs, bufr, buft, sems, semr, semo):
    wid = lax.axis_index("s") * NC + lax.axis_index("c")
    row0 = wid * E_PER_W
    lane = lax.iota(jnp.int32, 16)
    diags = [(lane + j) % 16 for j in range(D_OUT)]

    pltpu.sync_copy(eidx_hbm.at[0, pl.ds(row0, CHUNK)], idx_s.at[0])
    pltpu.sync_copy(eidx_hbm.at[1, pl.ds(row0, CHUNK)], idx_r.at[0])

    for ci in range(N_CHUNKS):
        p = ci % 2
        off = row0 + ci * CHUNK
        gs = pltpu.async_copy(xs_hbm.at[idx_s.at[p]], bufs, sems)
        gr = pltpu.async_copy(xr_hbm.at[idx_r.at[p]], bufr, semr)
        if ci + 1 < N_CHUNKS:
            off1 = off + CHUNK
            pltpu.sync_copy(eidx_hbm.at[0, pl.ds(off1, CHUNK)],
                            idx_s.at[1 - p])
            pltpu.sync_copy(eidx_hbm.at[1, pl.ds(off1, CHUNK)],
                            idx_r.at[1 - p])
        gs.wait()
        gr.wait()
        if ci > 0:
            drain = pltpu.make_async_copy(buft, out_hbm.at[:, pl.ds(0, CHUNK)],
                                          semo)
            drain.wait()

        def t_body(g, c):
            rows = g * 16 + lane
            vs = [plsc.load_gather(bufs, [rows, d])
                  + plsc.load_gather(bufr, [rows, d]) for d in diags]
            for j in range(D_OUT):
                plsc.store_scatter(buft, [diags[j], rows], vs[j])
            return c

        lax.fori_loop(0, CHUNK // 16, t_body, 0)
        pltpu.async_copy(buft, out_hbm.at[:, pl.ds(off, CHUNK)], semo)
    pltpu.make_async_copy(buft, out_hbm.at[:, pl.ds(0, CHUNK)], semo).wait()


def kernel(x, edge_index, edge_attr, W, b):
    w_sr = jnp.concatenate([W[:D_FEAT], W[D_FEAT:2 * D_FEAT]], axis=1)
    wet = W[2 * D_FEAT:].T
    bcol = b.reshape(D_OUT, 1)

    xs, xr = _node_proj(x, w_sr)
    tmpt = _sc_edge(xs, xr, edge_index)
    out_t = _final(edge_attr.T, tmpt, wet, bcol)
    return out_t.T


# speedup vs baseline: 13.8731x; 1.0516x over previous
import functools

import jax
import jax.numpy as jnp
from jax import lax
from jax.experimental import pallas as pl
from jax.experimental.pallas import tpu as pltpu
from jax.experimental.pallas import tpu_sc as plsc

N_NODES = 10000
N_EDGES = 320000
D_FEAT = 128
D_EDGE = 16
D_OUT = 16

NC, NS = 2, 16
NW = NC * NS
E_PER_W = N_EDGES // NW
CHUNK = 2000
N_CHUNKS = E_PER_W // CHUNK
SUB = 400
N_SUBS = CHUNK // SUB

FIN_BLK = 16000


def _proj_body(x_ref, w_ref, xs_ref, xr_ref):
    p = jnp.dot(x_ref[...], w_ref[...], preferred_element_type=jnp.float32)
    xs_ref[...] = p[:, :D_OUT]
    xr_ref[...] = p[:, D_OUT:]


def _node_proj(x, w_sr):
    return pl.pallas_call(
        _proj_body,
        out_shape=(
            jax.ShapeDtypeStruct((N_NODES, D_OUT), jnp.float32),
            jax.ShapeDtypeStruct((N_NODES, D_OUT), jnp.float32),
        ),
    )(x, w_sr)


def _final_body(eat_ref, tmpt_ref, wet_ref, b_ref, o_ref):
    o_ref[...] = (
        jnp.dot(wet_ref[...], eat_ref[...], preferred_element_type=jnp.float32)
        + b_ref[...]
        + tmpt_ref[...]
    )


def _final(eat, tmpt, wet, bcol):
    return pl.pallas_call(
        _final_body,
        grid=(N_EDGES // FIN_BLK,),
        in_specs=[
            pl.BlockSpec((D_OUT, FIN_BLK), lambda i: (0, i)),
            pl.BlockSpec((D_OUT, FIN_BLK), lambda i: (0, i)),
            pl.BlockSpec((D_EDGE, D_OUT), lambda i: (0, 0)),
            pl.BlockSpec((D_OUT, 1), lambda i: (0, 0)),
        ],
        out_specs=pl.BlockSpec((D_OUT, FIN_BLK), lambda i: (0, i)),
        out_shape=jax.ShapeDtypeStruct((D_OUT, N_EDGES), jnp.float32),
    )(eat, tmpt, wet, bcol)


_SC_MESH = plsc.VectorSubcoreMesh(core_axis_name="c", subcore_axis_name="s")


@functools.partial(
    pl.kernel,
    out_type=jax.ShapeDtypeStruct((D_OUT, N_EDGES), jnp.float32),
    mesh=_SC_MESH,
    compiler_params=pltpu.CompilerParams(
        use_tc_tiling_on_sc=False, needs_layout_passes=False),
    scratch_types=[
        pltpu.VMEM((2, CHUNK), jnp.int32),
        pltpu.VMEM((2, CHUNK), jnp.int32),
        pltpu.VMEM((2, SUB, D_OUT), jnp.float32),
        pltpu.VMEM((2, SUB, D_OUT), jnp.float32),
        pltpu.VMEM((D_OUT, CHUNK), jnp.float32),
        pltpu.SemaphoreType.DMA,
        pltpu.SemaphoreType.DMA,
        pltpu.SemaphoreType.DMA,
    ],
)
def _sc_edge(xs_hbm, xr_hbm, eidx_hbm, out_hbm,
             idx_s, idx_r, bufs, bufr, buft, sems, semr, semo):
    wid = lax.axis_index("s") * NC + lax.axis_index("c")
    row0 = wid * E_PER_W
    lane = lax.iota(jnp.int32, 16)
    diags = [(lane + j) % 16 for j in range(D_OUT)]

    def issue_gathers(ci, si):
        ip = ci % 2
        sp = si % 2
        sl = pl.ds(si * SUB, SUB)
        pltpu.async_copy(xs_hbm.at[idx_s.at[ip, sl]], bufs.at[sp], sems)
        pltpu.async_copy(xr_hbm.at[idx_r.at[ip, sl]], bufr.at[sp], semr)

    def wait_gathers(si):
        sp = si % 2
        pltpu.make_async_copy(xs_hbm.at[pl.ds(0, SUB)], bufs.at[sp], sems).wait()
        pltpu.make_async_copy(xr_hbm.at[pl.ds(0, SUB)], bufr.at[sp], semr).wait()

    pltpu.sync_copy(eidx_hbm.at[0, pl.ds(row0, CHUNK)], idx_s.at[0])
    pltpu.sync_copy(eidx_hbm.at[1, pl.ds(row0, CHUNK)], idx_r.at[0])
    issue_gathers(0, 0)

    for ci in range(N_CHUNKS):
        p = ci % 2
        off = row0 + ci * CHUNK
        if ci + 1 < N_CHUNKS:
            off1 = off + CHUNK
            pltpu.sync_copy(eidx_hbm.at[0, pl.ds(off1, CHUNK)],
                            idx_s.at[1 - p])
            pltpu.sync_copy(eidx_hbm.at[1, pl.ds(off1, CHUNK)],
                            idx_r.at[1 - p])
        if ci > 0:
            pltpu.make_async_copy(buft, out_hbm.at[:, pl.ds(0, CHUNK)],
                                  semo).wait()
        for si in range(N_SUBS):
            sp = si % 2
            wait_gathers(si)
            if si + 1 < N_SUBS:
                issue_gathers(ci, si + 1)
            elif ci + 1 < N_CHUNKS:
                issue_gathers(ci + 1, 0)

            def t_body(g, c, _sp=sp, _si=si):
                rows = g * 16 + lane
                vs = [plsc.load_gather(bufs.at[_sp], [rows, d])
                      + plsc.load_gather(bufr.at[_sp], [rows, d])
                      for d in diags]
                cols = _si * SUB + rows
                for j in range(D_OUT):
                    plsc.store_scatter(buft, [diags[j], cols], vs[j])
                return c

            lax.fori_loop(0, SUB // 16, t_body, 0)
        pltpu.async_copy(buft, out_hbm.at[:, pl.ds(off, CHUNK)], semo)
    pltpu.make_async_copy(buft, out_hbm.at[:, pl.ds(0, CHUNK)], semo).wait()


def kernel(x, edge_index, edge_attr, W, b):
    w_sr = jnp.concatenate([W[:D_FEAT], W[D_FEAT:2 * D_FEAT]], axis=1)
    wet = W[2 * D_FEAT:].T
    bcol = b.reshape(D_OUT, 1)

    xs, xr = _node_proj(x, w_sr)
    tmpt = _sc_edge(xs, xr, edge_index)
    out_t = _final(edge_attr.T, tmpt, wet, bcol)
    return out_t.T


# speedup vs baseline: 15.5075x; 1.1178x over previous
import functools

import jax
import jax.numpy as jnp
from jax import lax
from jax.experimental import pallas as pl
from jax.experimental.pallas import tpu as pltpu
from jax.experimental.pallas import tpu_sc as plsc

N_NODES = 10000
N_EDGES = 320000
D_FEAT = 128
D_EDGE = 16
D_OUT = 16

NC, NS = 2, 16
NW = NC * NS
E_PER_W = N_EDGES // NW
CHUNK = 2000
N_CHUNKS = E_PER_W // CHUNK
SUB = 400
N_SUBS = CHUNK // SUB

FIN_BLK = 16384


def _proj_body(x_ref, w_ref, xs_ref, xr_ref):
    p = jnp.dot(x_ref[...], w_ref[...], preferred_element_type=jnp.float32)
    xs_ref[...] = p[:, :D_OUT]
    xr_ref[...] = p[:, D_OUT:]


def _node_proj(x, w_sr):
    return pl.pallas_call(
        _proj_body,
        out_shape=(
            jax.ShapeDtypeStruct((N_NODES, D_OUT), jnp.float32),
            jax.ShapeDtypeStruct((N_NODES, D_OUT), jnp.float32),
        ),
    )(x, w_sr)


def _final_body(eat_ref, wet_ref, b_ref, *refs):
    o_ref = refs[-1]
    base = (
        jnp.dot(wet_ref[...], eat_ref[...], preferred_element_type=jnp.float32)
        + b_ref[...]
    )
    for j in range(D_OUT):
        o_ref[j:j + 1, :] = base[j:j + 1, :] + refs[j][...][None, :]


def _final(eat, wet, bcol, tmps):
    return pl.pallas_call(
        _final_body,
        grid=(pl.cdiv(N_EDGES, FIN_BLK),),
        in_specs=[
            pl.BlockSpec((D_OUT, FIN_BLK), lambda i: (0, i)),
            pl.BlockSpec((D_EDGE, D_OUT), lambda i: (0, 0)),
            pl.BlockSpec((D_OUT, 1), lambda i: (0, 0)),
        ] + [pl.BlockSpec((FIN_BLK,), lambda i: (i,)) for _ in range(D_OUT)],
        out_specs=pl.BlockSpec((D_OUT, FIN_BLK), lambda i: (0, i)),
        out_shape=jax.ShapeDtypeStruct((D_OUT, N_EDGES), jnp.float32),
    )(eat, wet, bcol, *tmps)


_SC_MESH = plsc.VectorSubcoreMesh(core_axis_name="c", subcore_axis_name="s")


@functools.partial(
    pl.kernel,
    out_type=tuple(jax.ShapeDtypeStruct((N_EDGES,), jnp.float32)
                   for _ in range(D_OUT)),
    mesh=_SC_MESH,
    compiler_params=pltpu.CompilerParams(
        use_tc_tiling_on_sc=False, needs_layout_passes=False),
    scratch_types=[
        pltpu.VMEM((2, CHUNK), jnp.int32),
        pltpu.VMEM((2, CHUNK), jnp.int32),
        pltpu.VMEM((2, SUB, D_OUT), jnp.float32),
        pltpu.VMEM((2, SUB, D_OUT), jnp.float32),
        pltpu.VMEM((D_OUT, CHUNK), jnp.float32),
        pltpu.SemaphoreType.DMA,
        pltpu.SemaphoreType.DMA,
        pltpu.SemaphoreType.DMA,
    ],
)
def _sc_edge(xs_hbm, xr_hbm, eidx_hbm, *refs):
    out_hbms = refs[:D_OUT]
    (idx_s, idx_r, bufs, bufr, buft, sems, semr, semo) = refs[D_OUT:]
    wid = lax.axis_index("s") * NC + lax.axis_index("c")
    row0 = wid * E_PER_W
    lane = lax.iota(jnp.int32, 16)
    diags = [(lane + j) % 16 for j in range(D_OUT)]

    def issue_gathers(ci, si):
        ip = ci % 2
        sp = si % 2
        sl = pl.ds(si * SUB, SUB)
        pltpu.async_copy(xs_hbm.at[idx_s.at[ip, sl]], bufs.at[sp], sems)
        pltpu.async_copy(xr_hbm.at[idx_r.at[ip, sl]], bufr.at[sp], semr)

    def wait_gathers(si):
        sp = si % 2
        pltpu.make_async_copy(xs_hbm.at[pl.ds(0, SUB)], bufs.at[sp], sems).wait()
        pltpu.make_async_copy(xr_hbm.at[pl.ds(0, SUB)], bufr.at[sp], semr).wait()

    pltpu.sync_copy(eidx_hbm.at[0, pl.ds(row0, CHUNK)], idx_s.at[0])
    pltpu.sync_copy(eidx_hbm.at[1, pl.ds(row0, CHUNK)], idx_r.at[0])
    issue_gathers(0, 0)

    for ci in range(N_CHUNKS):
        p = ci % 2
        off = row0 + ci * CHUNK
        if ci + 1 < N_CHUNKS:
            off1 = off + CHUNK
            pltpu.sync_copy(eidx_hbm.at[0, pl.ds(off1, CHUNK)],
                            idx_s.at[1 - p])
            pltpu.sync_copy(eidx_hbm.at[1, pl.ds(off1, CHUNK)],
                            idx_r.at[1 - p])
        if ci > 0:
            for j in range(D_OUT):
                pltpu.make_async_copy(buft.at[j],
                                      out_hbms[j].at[pl.ds(0, CHUNK)],
                                      semo).wait()
        for si in range(N_SUBS):
            sp = si % 2
            wait_gathers(si)
            if si + 1 < N_SUBS:
                issue_gathers(ci, si + 1)
            elif ci + 1 < N_CHUNKS:
                issue_gathers(ci + 1, 0)

            def t_body(g, c, _sp=sp, _si=si):
                rows = g * 16 + lane
                vs = [plsc.load_gather(bufs.at[_sp], [rows, d])
                      + plsc.load_gather(bufr.at[_sp], [rows, d])
                      for d in diags]
                cols = _si * SUB + rows
                for j in range(D_OUT):
                    plsc.store_scatter(buft, [diags[j], cols], vs[j])
                return c

            lax.fori_loop(0, SUB // 16, t_body, 0)
        for j in range(D_OUT):
            pltpu.async_copy(buft.at[j], out_hbms[j].at[pl.ds(off, CHUNK)],
                             semo)
    for j in range(D_OUT):
        pltpu.make_async_copy(buft.at[j], out_hbms[j].at[pl.ds(0, CHUNK)],
                              semo).wait()


def kernel(x, edge_index, edge_attr, W, b):
    w_sr = jnp.concatenate([W[:D_FEAT], W[D_FEAT:2 * D_FEAT]], axis=1)
    wet = W[2 * D_FEAT:].T
    bcol = b.reshape(D_OUT, 1)

    xs, xr = _node_proj(x, w_sr)
    tmps = _sc_edge(xs, xr, edge_index)
    out_t = _final(edge_attr.T, wet, bcol, tmps)
    return out_t.T
